# Initial kernel scaffold; baseline (speedup 1.0000x reference)
import jax
import jax.numpy as jnp
from jax.experimental import pallas as pl


def kernel(input_ids, table):
    raise NotImplementedError("write your pallas kernel here")


# speedup vs baseline: 1.8459x; 1.8459x over previous
import functools

import jax
import jax.numpy as jnp
from jax import lax
from jax.experimental import pallas as pl
from jax.experimental.pallas import tpu as pltpu
from jax.experimental.pallas import tpu_sc as plsc

_NUM_WORKERS = 32
_CHUNK = 1024


def _gather_body(table_hbm, idx_hbm, out_hbm, idx_v, rows_v, sem):
    n_total = idx_hbm.shape[0]
    per_w = n_total // _NUM_WORKERS
    n_chunks = per_w // _CHUNK
    wid = lax.axis_index("s") * 2 + lax.axis_index("c")
    base = wid * per_w

    @pl.loop(0, n_chunks)
    def _(k):
        off = base + k * _CHUNK
        pltpu.sync_copy(idx_hbm.at[pl.ds(off, _CHUNK)], idx_v)
        pltpu.async_copy(table_hbm.at[idx_v], rows_v, sem).wait()
        pltpu.sync_copy(rows_v, out_hbm.at[pl.ds(off, _CHUNK)])


@jax.jit
def _embedding_gather(table, idx):
    n = idx.shape[0]
    d = table.shape[1]
    mesh = plsc.VectorSubcoreMesh(core_axis_name="c", subcore_axis_name="s")
    f = pl.kernel(
        _gather_body,
        out_type=jax.ShapeDtypeStruct((n, d), table.dtype),
        mesh=mesh,
        scratch_types=[
            pltpu.VMEM((_CHUNK,), jnp.int32),
            pltpu.VMEM((_CHUNK, d), jnp.float32),
            pltpu.SemaphoreType.DMA,
        ],
        compiler_params=pltpu.CompilerParams(use_tc_tiling_on_sc=False),
    )
    return f(table, idx)


def kernel(input_ids, table):
    batch, hist = input_ids.shape
    idx = input_ids.reshape(-1).astype(jnp.int32)
    out = _embedding_gather(table, idx)
    return out.reshape(batch, hist, table.shape[1])


# speedup vs baseline: 1.8755x; 1.0161x over previous
import jax
import jax.numpy as jnp
from jax import lax
from jax.experimental import pallas as pl
from jax.experimental.pallas import tpu as pltpu
from jax.experimental.pallas import tpu_sc as plsc

_NUM_WORKERS = 32
_CHUNK = 800
_NBUF = 2


def _gather_body(table_hbm, idx_hbm, out_hbm, idx_v, rows0, rows1,
                 sg0, sg1, so0, so1):
    n_total = idx_hbm.shape[0]
    per_w = n_total // _NUM_WORKERS
    n_chunks = per_w // _CHUNK
    wid = lax.axis_index("s") * 2 + lax.axis_index("c")
    base = wid * per_w

    rows = (rows0, rows1)
    sg = (sg0, sg1)
    so = (so0, so1)

    pltpu.sync_copy(idx_hbm.at[pl.ds(base, per_w)], idx_v)

    def gather_start(chunk, b):
        pltpu.async_copy(
            table_hbm.at[idx_v.at[pl.ds(chunk * _CHUNK, _CHUNK)]],
            rows[b], sg[b])

    def gather_wait(b):
        pltpu.make_async_copy(
            table_hbm.at[idx_v.at[pl.ds(0, _CHUNK)]], rows[b], sg[b]).wait()

    def out_start(chunk, b):
        pltpu.async_copy(
            rows[b], out_hbm.at[pl.ds(base + chunk * _CHUNK, _CHUNK)], so[b])

    def out_wait(b):
        pltpu.make_async_copy(
            rows[b], out_hbm.at[pl.ds(base, _CHUNK)], so[b]).wait()

    for b in range(_NBUF):
        gather_start(b, b)

    @pl.loop(0, n_chunks - _NBUF, step=_NBUF)
    def _(k):
        for b in range(_NBUF):
            c = k + b
            gather_wait(b)
            out_start(c, b)
            out_wait(b)
            gather_start(c + _NBUF, b)

    for b in range(_NBUF):
        gather_wait(b)
        out_start(n_chunks - _NBUF + b, b)
    for b in range(_NBUF):
        out_wait(b)


@jax.jit
def _embedding_gather(table, idx):
    n = idx.shape[0]
    d = table.shape[1]
    per_w = n // _NUM_WORKERS
    mesh = plsc.VectorSubcoreMesh(core_axis_name="c", subcore_axis_name="s")
    f = pl.kernel(
        _gather_body,
        out_type=jax.ShapeDtypeStruct((n, d), table.dtype),
        mesh=mesh,
        scratch_types=[
            pltpu.VMEM((per_w,), jnp.int32),
            pltpu.VMEM((_CHUNK, d), jnp.float32),
            pltpu.VMEM((_CHUNK, d), jnp.float32),
            pltpu.SemaphoreType.DMA,
            pltpu.SemaphoreType.DMA,
            pltpu.SemaphoreType.DMA,
            pltpu.SemaphoreType.DMA,
        ],
        compiler_params=pltpu.CompilerParams(use_tc_tiling_on_sc=False),
    )
    return f(table, idx)


def kernel(input_ids, table):
    batch, hist = input_ids.shape
    idx = input_ids.reshape(-1).astype(jnp.int32)
    out = _embedding_gather(table, idx)
    return out.reshape(batch, hist, table.shape[1])
